# baseline (speedup 1.0000x reference)
import jax
import jax.numpy as jnp
from jax import lax
from jax.experimental import pallas as pl
from jax.experimental.pallas import tpu as pltpu

EMB = 1024
NPOOL = 8192
TOPK = 8
NBLK = 8
ROWS = NPOOL // NBLK

BATCH = 4096
BBLK = 256
NB = BATCH // BBLK


def _fused_kernel(q_ref, keys_ref, x_ref, prompts_hbm, o_ref,
                  sims_ref, sel_ref, sem):
    i = pl.program_id(0)

    @pl.when(i < NBLK)
    def _sims():
        kb = keys_ref[...]
        q = q_ref[...]
        kq = jnp.dot(kb, q.T, preferred_element_type=jnp.float32)
        kn = jnp.sqrt(jnp.sum(kb * kb, axis=1, keepdims=True))
        qn = jnp.sqrt(jnp.sum(q * q))
        sims = kq[:, 0] / jnp.maximum(kn[:, 0] * qn, 1e-8)
        sims_ref[i, :] = sims.reshape(1, ROWS)[0, :]

    @pl.when(i == NBLK - 1)
    def _topk_gather():
        s = sims_ref[...]
        fidx = (lax.broadcasted_iota(jnp.int32, (NBLK, ROWS), 0) * ROWS
                + lax.broadcasted_iota(jnp.int32, (NBLK, ROWS), 1))
        copies = []
        for k in range(TOPK):
            m = jnp.max(s)
            cand = jnp.where(s == m, fidx, jnp.int32(2 ** 30))
            idx = jnp.min(cand)
            s = jnp.where(fidx == idx, -jnp.inf, s)
            c = pltpu.make_async_copy(
                prompts_hbm.at[pl.ds(idx, 1), :],
                sel_ref.at[pl.ds(k, 1), :],
                sem,
            )
            c.start()
            copies.append(c)
        for c in copies:
            c.wait()

    @pl.when(i >= NBLK)
    def _add():
        o_ref[...] = x_ref[...] + sel_ref[...][None, :, :]


@jax.jit
def kernel(x, query, prompts, keys):
    q2 = query.reshape(1, EMB)
    out = pl.pallas_call(
        _fused_kernel,
        grid=(NBLK + NB,),
        in_specs=[
            pl.BlockSpec((1, EMB), lambda i: (0, 0)),
            pl.BlockSpec((ROWS, EMB), lambda i: (jnp.minimum(i, NBLK - 1), 0)),
            pl.BlockSpec((BBLK, TOPK, EMB),
                         lambda i: (jnp.maximum(i - NBLK, 0), 0, 0)),
            pl.BlockSpec(memory_space=pl.ANY),
        ],
        out_specs=pl.BlockSpec((BBLK, TOPK, EMB),
                               lambda i: (jnp.maximum(i - NBLK, 0), 0, 0)),
        out_shape=jax.ShapeDtypeStruct((BATCH, TOPK, EMB), jnp.float32),
        scratch_shapes=[
            pltpu.VMEM((NBLK, ROWS), jnp.float32),
            pltpu.VMEM((TOPK, EMB), jnp.float32),
            pltpu.SemaphoreType.DMA,
        ],
    )(q2, keys, x, prompts)
    return out


# speedup vs baseline: 1.0894x; 1.0894x over previous
import jax
import jax.numpy as jnp
from jax import lax
from jax.experimental import pallas as pl
from jax.experimental.pallas import tpu as pltpu

EMB = 1024
NPOOL = 8192
TOPK = 8
NBLK = 8
ROWS = NPOOL // NBLK

BATCH = 4096
BBLK = 128
NB = BATCH // BBLK
RING = NBLK + 1


def _fused_kernel(q_ref, keys_ref, x_hbm, prompts_hbm, o_ref,
                  sims_ref, sel_ref, x_bufs, gsem, xsems):
    i = pl.program_id(0)

    @pl.when(i < NB)
    def _issue():
        slot = lax.rem(i, RING)
        pltpu.make_async_copy(
            x_hbm.at[pl.ds(i * BBLK, BBLK)],
            x_bufs.at[slot],
            xsems.at[slot],
        ).start()

    @pl.when(i < NBLK)
    def _sims():
        kb = keys_ref[...]
        q = q_ref[...]
        kq = jnp.dot(kb, q.T, preferred_element_type=jnp.float32)
        kn = jnp.sqrt(jnp.sum(kb * kb, axis=1, keepdims=True))
        qn = jnp.sqrt(jnp.sum(q * q))
        sims = kq[:, 0] / jnp.maximum(kn[:, 0] * qn, 1e-8)
        sims_ref[i, :] = sims.reshape(1, ROWS)[0, :]

    @pl.when(i == NBLK - 1)
    def _topk_gather():
        s = sims_ref[...]
        fidx = (lax.broadcasted_iota(jnp.int32, (NBLK, ROWS), 0) * ROWS
                + lax.broadcasted_iota(jnp.int32, (NBLK, ROWS), 1))
        copies = []
        for k in range(TOPK):
            m = jnp.max(s)
            cand = jnp.where(s == m, fidx, jnp.int32(2 ** 30))
            idx = jnp.min(cand)
            s = jnp.where(fidx == idx, -jnp.inf, s)
            c = pltpu.make_async_copy(
                prompts_hbm.at[pl.ds(idx, 1), :],
                sel_ref.at[pl.ds(k, 1), :],
                gsem,
            )
            c.start()
            copies.append(c)
        for c in copies:
            c.wait()

    @pl.when(i >= NBLK)
    def _add():
        b = i - NBLK
        slot = lax.rem(b, RING)
        pltpu.make_async_copy(
            x_hbm.at[pl.ds(b * BBLK, BBLK)],
            x_bufs.at[slot],
            xsems.at[slot],
        ).wait()
        o_ref[...] = x_bufs[slot] + sel_ref[...][None, :, :]


@jax.jit
def kernel(x, query, prompts, keys):
    q2 = query.reshape(1, EMB)
    out = pl.pallas_call(
        _fused_kernel,
        grid=(NBLK + NB,),
        in_specs=[
            pl.BlockSpec((1, EMB), lambda i: (0, 0)),
            pl.BlockSpec((ROWS, EMB), lambda i: (jnp.minimum(i, NBLK - 1), 0)),
            pl.BlockSpec(memory_space=pl.ANY),
            pl.BlockSpec(memory_space=pl.ANY),
        ],
        out_specs=pl.BlockSpec((BBLK, TOPK, EMB),
                               lambda i: (jnp.maximum(i - NBLK, 0), 0, 0)),
        out_shape=jax.ShapeDtypeStruct((BATCH, TOPK, EMB), jnp.float32),
        scratch_shapes=[
            pltpu.VMEM((NBLK, ROWS), jnp.float32),
            pltpu.VMEM((TOPK, EMB), jnp.float32),
            pltpu.VMEM((RING, BBLK, TOPK, EMB), jnp.float32),
            pltpu.SemaphoreType.DMA,
            pltpu.SemaphoreType.DMA((RING,)),
        ],
    )(q2, keys, x, prompts)
    return out
